# baseline (speedup 1.0000x reference)
import functools

import jax
import jax.numpy as jnp
from jax import lax
from jax.experimental import pallas as pl
from jax.experimental.pallas import tpu as pltpu
from jax.experimental.pallas import tpu_sc as plsc

DM = 1024
DS = 64
NE = 8
NA = 2
FH = 2048
CK = 64
EPS = 1e-8
GBLK = 256


def _compnorm(x, tau, scale):
    rms = jax.lax.rsqrt(jnp.mean(x * x, axis=-1, keepdims=True) + EPS)
    x_norm = x * rms
    xc = x - jnp.mean(x, axis=-1, keepdims=True)
    gate = jax.nn.softmax(xc / jnp.maximum(tau, 1.0), axis=-1)
    return x_norm * gate * scale * DM


def _k1(x_ref, tau_ref, scale_ref, wa_ref, wg_ref, ba_ref, bg_ref, u_ref):
    x = x_ref[...]
    h = _compnorm(x, tau_ref[0, 0], scale_ref[...])
    iv = jnp.tanh(jnp.dot(h, wa_ref[...], preferred_element_type=jnp.float32)
                  + ba_ref[...])
    g = jax.nn.sigmoid(jnp.dot(h, wg_ref[...], preferred_element_type=jnp.float32)
                       + bg_ref[...])
    u_ref[...] = g * iv


def _k2(ut_ref, u_ref, rec_ref, pos_ref, state_ref,
        fu_ref, hs_ref, ns_ref, r_scr, *, nb):
    la_col = jnp.log(jax.nn.sigmoid(rec_ref[...] + pos_ref[...]))
    la_row = la_col.reshape(1, DS)
    i_col = jax.lax.broadcasted_iota(jnp.int32, (CK, 1), 0).astype(jnp.float32)
    j2 = jax.lax.broadcasted_iota(jnp.int32, (CK, CK), 0).astype(jnp.float32)
    i2 = jax.lax.broadcasted_iota(jnp.int32, (CK, CK), 1).astype(jnp.float32)
    expn = jnp.maximum(j2 - i2, 0.0)
    mask = (j2 >= i2).astype(jnp.float32)
    L = jnp.exp(la_col.reshape(DS, 1, 1) * expn[None]) * mask[None]
    fu = jax.lax.dot_general(L, ut_ref[...],
                             (((2,), (1,)), ((0,), (0,))),
                             preferred_element_type=jnp.float32)
    fu_ref[...] = fu
    w = jnp.exp(la_row * (CK - 1.0 - i_col))
    r_scr[...] = jnp.sum(u_ref[...] * w[None], axis=1)
    a_ck = jnp.exp(la_row * float(CK))

    bsz = state_ref.shape[0]

    def body(c, h_cur):
        for b in range(bsz):
            hs_ref[pl.ds(b * nb + c, 1), :] = h_cur[b:b + 1]
        rows = jnp.concatenate(
            [r_scr[pl.ds(b * nb + c, 1), :] for b in range(bsz)], axis=0)
        return h_cur * a_ck + rows

    h_fin = jax.lax.fori_loop(0, nb, body, state_ref[...])
    ns_ref[...] = h_fin


def _k3(fu_ref, hs_ref, ap_ref, x_ref, tau_ref, scale_ref, wout_ref,
        bout_ref, wr_ref, x2_ref, h3_ref, io_ref, wc_ref, ps_ref, cs_ref):
    states = fu_ref[...] + hs_ref[...] * ap_ref[...]
    h2 = jnp.dot(states, wout_ref[...], preferred_element_type=jnp.float32) \
        + bout_ref[...]
    x2 = x_ref[...] + h2
    x2_ref[...] = x2
    h3 = _compnorm(x2, tau_ref[0, 0], scale_ref[...])
    h3_ref[...] = h3
    logits = jnp.dot(h3, wr_ref[...], preferred_element_type=jnp.float32)
    probs = jax.nn.softmax(logits, axis=-1)
    eio = jax.lax.broadcasted_iota(jnp.int32, logits.shape, 1)
    m1 = jnp.max(logits, axis=-1, keepdims=True)
    idx1 = jnp.min(jnp.where(logits == m1, eio, NE), axis=-1, keepdims=True)
    mask1 = (eio == idx1)
    ml = jnp.where(mask1, -jnp.inf, logits)
    m2 = jnp.max(ml, axis=-1, keepdims=True)
    idx2 = jnp.min(jnp.where(ml == m2, eio, NE), axis=-1, keepdims=True)
    mask2 = (eio == idx2)
    p1 = jnp.sum(jnp.where(mask1, probs, 0.0), axis=-1, keepdims=True)
    p2 = jnp.sum(jnp.where(mask2, probs, 0.0), axis=-1, keepdims=True)
    inv = 1.0 / (p1 + p2)
    two = jax.lax.broadcasted_iota(jnp.int32, (logits.shape[0], 2), 1)
    io_ref[...] = jnp.where(two == 0, idx1, idx2)
    wc_ref[...] = jnp.where(two == 0, p1 * inv, p2 * inv)
    ps_ref[...] = jnp.sum(probs, axis=0, keepdims=True)[None]
    cs_ref[...] = jnp.sum(mask1.astype(jnp.float32) + mask2.astype(jnp.float32),
                          axis=0, keepdims=True)[None]


def _sc_gather(table, idx, rpc):
    sci = plsc.get_sparse_core_info()
    ncores = sci.num_cores
    nw = ncores * sci.num_subcores
    n = idx.shape[0]
    d = table.shape[1]
    b_per_w = n // nw
    nchunk = b_per_w // rpc
    mesh = plsc.VectorSubcoreMesh(core_axis_name="c", subcore_axis_name="s")

    @functools.partial(
        pl.kernel, mesh=mesh,
        out_type=jax.ShapeDtypeStruct((n, d), table.dtype),
        scratch_types=[
            pltpu.VMEM((rpc,), jnp.int32),
            pltpu.VMEM((rpc, d), table.dtype),
            pltpu.SemaphoreType.DMA,
        ],
    )
    def k(table_hbm, idx_hbm, out_hbm, idx_v, rows_v, sem):
        wid = lax.axis_index("s") * ncores + lax.axis_index("c")
        base = wid * b_per_w

        @pl.loop(0, nchunk)
        def body(c):
            off = base + c * rpc
            pltpu.sync_copy(idx_hbm.at[pl.ds(off, rpc)], idx_v)
            pltpu.async_copy(table_hbm.at[idx_v], rows_v, sem).wait()
            pltpu.sync_copy(rows_v, out_hbm.at[pl.ds(off, rpc)])

    return k(table, idx)


def _k5(be_ref, xg_ref, w1_ref, b1_ref, w2_ref, b2_ref, y_ref):
    h = jax.nn.silu(jnp.dot(xg_ref[...], w1_ref[0],
                            preferred_element_type=jnp.float32) + b1_ref[0])
    y_ref[...] = jnp.dot(h, w2_ref[0],
                         preferred_element_type=jnp.float32) + b2_ref[0]


def _k6(x2_ref, ya_ref, yb_ref, wc_ref, out_ref):
    w = wc_ref[...]
    out_ref[...] = (x2_ref[...] + w[:, 0:1] * ya_ref[...]
                    + w[:, 1:2] * yb_ref[...])


def kernel(x, state, tau1, scale1, tau2, scale2, Win, bin_, rec_w, pos_bias,
           Wout, bout, Wr, W1, b1, W2, b2):
    B, T, D = x.shape
    ds = Wout.shape[0]
    ne = Wr.shape[1]
    fh = W1.shape[2]
    BT = B * T
    nch = T // CK
    BC = B * nch
    xf = x.reshape(BT, D)
    f32 = jnp.float32

    blk1 = min(512, BT)
    n1 = BT // blk1
    u = pl.pallas_call(
        _k1,
        grid=(n1,),
        in_specs=[
            pl.BlockSpec((blk1, D), lambda i: (i, 0)),
            pl.BlockSpec((1, 1), lambda i: (0, 0)),
            pl.BlockSpec((1, D), lambda i: (0, 0)),
            pl.BlockSpec((D, ds), lambda i: (0, 0)),
            pl.BlockSpec((D, ds), lambda i: (0, 0)),
            pl.BlockSpec((1, ds), lambda i: (0, 0)),
            pl.BlockSpec((1, ds), lambda i: (0, 0)),
        ],
        out_specs=pl.BlockSpec((blk1, ds), lambda i: (i, 0)),
        out_shape=jax.ShapeDtypeStruct((BT, ds), f32),
    )(xf, tau1.reshape(1, 1), scale1.reshape(1, D), Win[:, :ds], Win[:, ds:],
      bin_[:ds].reshape(1, ds), bin_[ds:].reshape(1, ds))

    u4 = u.reshape(B, nch, CK, ds)
    ut = jnp.transpose(u4, (3, 2, 0, 1)).reshape(ds, CK, BC)
    uo = u4.reshape(BC, CK, ds)

    fu, hs, new_state = pl.pallas_call(
        functools.partial(_k2, nb=nch),
        grid=(1,),
        in_specs=[
            pl.BlockSpec((ds, CK, BC), lambda i: (0, 0, 0)),
            pl.BlockSpec((BC, CK, ds), lambda i: (0, 0, 0)),
            pl.BlockSpec((ds, 1), lambda i: (0, 0)),
            pl.BlockSpec((ds, 1), lambda i: (0, 0)),
            pl.BlockSpec((B, ds), lambda i: (0, 0)),
        ],
        out_specs=[
            pl.BlockSpec((ds, CK, BC), lambda i: (0, 0, 0)),
            pl.BlockSpec((BC, ds), lambda i: (0, 0)),
            pl.BlockSpec((B, ds), lambda i: (0, 0)),
        ],
        out_shape=[
            jax.ShapeDtypeStruct((ds, CK, BC), f32),
            jax.ShapeDtypeStruct((BC, ds), f32),
            jax.ShapeDtypeStruct((B, ds), f32),
        ],
        scratch_shapes=[pltpu.VMEM((BC, ds), f32)],
    )(ut, uo, rec_w.reshape(ds, 1), pos_bias.reshape(ds, 1), state)

    fu_t = jnp.transpose(fu, (2, 1, 0)).reshape(BT, ds)
    hs_full = jnp.repeat(hs, CK, axis=0)
    a = jax.nn.sigmoid(rec_w + pos_bias)
    jj = (jnp.arange(CK, dtype=f32) + 1.0)[:, None]
    apow = a[None, :] ** jj
    ap_full = jnp.tile(apow, (BC, 1))

    blk3 = min(512, BT)
    n3 = BT // blk3
    x2, h3, iout, wcol, psum, csum = pl.pallas_call(
        _k3,
        grid=(n3,),
        in_specs=[
            pl.BlockSpec((blk3, ds), lambda i: (i, 0)),
            pl.BlockSpec((blk3, ds), lambda i: (i, 0)),
            pl.BlockSpec((blk3, ds), lambda i: (i, 0)),
            pl.BlockSpec((blk3, D), lambda i: (i, 0)),
            pl.BlockSpec((1, 1), lambda i: (0, 0)),
            pl.BlockSpec((1, D), lambda i: (0, 0)),
            pl.BlockSpec((ds, D), lambda i: (0, 0)),
            pl.BlockSpec((1, D), lambda i: (0, 0)),
            pl.BlockSpec((D, ne), lambda i: (0, 0)),
        ],
        out_specs=[
            pl.BlockSpec((blk3, D), lambda i: (i, 0)),
            pl.BlockSpec((blk3, D), lambda i: (i, 0)),
            pl.BlockSpec((blk3, NA), lambda i: (i, 0)),
            pl.BlockSpec((blk3, NA), lambda i: (i, 0)),
            pl.BlockSpec((1, 1, ne), lambda i: (i, 0, 0)),
            pl.BlockSpec((1, 1, ne), lambda i: (i, 0, 0)),
        ],
        out_shape=[
            jax.ShapeDtypeStruct((BT, D), f32),
            jax.ShapeDtypeStruct((BT, D), f32),
            jax.ShapeDtypeStruct((BT, NA), jnp.int32),
            jax.ShapeDtypeStruct((BT, NA), f32),
            jax.ShapeDtypeStruct((n3, 1, ne), f32),
            jax.ShapeDtypeStruct((n3, 1, ne), f32),
        ],
    )(fu_t, hs_full, ap_full, xf, tau2.reshape(1, 1), scale2.reshape(1, D),
      Wout, bout.reshape(1, D), Wr)

    npair = BT * NA
    nblk = npair // GBLK + NE
    npad = nblk * GBLK
    ep = iout.reshape(-1)
    perm = jnp.argsort(ep, stable=True).astype(jnp.int32)
    ep_sorted = ep[perm]
    counts = jnp.sum((ep[:, None] == jnp.arange(ne)[None, :]), axis=0)
    nbe = (counts + GBLK - 1) // GBLK
    nbc = jnp.cumsum(nbe)
    pstart = ((nbc - nbe) * GBLK).astype(jnp.int32)
    cstart = (jnp.cumsum(counts) - counts).astype(jnp.int32)
    jpos = jnp.arange(npair, dtype=jnp.int32)
    pp = pstart[ep_sorted] + jpos - cstart[ep_sorted]
    tok_for_row = jnp.zeros((npad,), jnp.int32).at[pp].set(perm // NA)
    pos_of_pair = jnp.zeros((npair,), jnp.int32).at[perm].set(pp)
    pair_pos = jnp.concatenate([pos_of_pair[0::2], pos_of_pair[1::2]])
    block_expert = jnp.minimum(
        jnp.searchsorted(nbc, jnp.arange(nblk), side='right'),
        ne - 1).astype(jnp.int32)

    xg = _sc_gather(h3, tok_for_row, 32)

    grid_spec = pltpu.PrefetchScalarGridSpec(
        num_scalar_prefetch=1,
        grid=(nblk,),
        in_specs=[
            pl.BlockSpec((GBLK, D), lambda i, be: (i, 0)),
            pl.BlockSpec((1, D, fh), lambda i, be: (be[i], 0, 0)),
            pl.BlockSpec((1, 1, fh), lambda i, be: (be[i], 0, 0)),
            pl.BlockSpec((1, fh, D), lambda i, be: (be[i], 0, 0)),
            pl.BlockSpec((1, 1, D), lambda i, be: (be[i], 0, 0)),
        ],
        out_specs=pl.BlockSpec((GBLK, D), lambda i, be: (i, 0)),
    )
    y = pl.pallas_call(
        _k5,
        grid_spec=grid_spec,
        out_shape=jax.ShapeDtypeStruct((npad, D), f32),
    )(block_expert, xg, W1, b1.reshape(ne, 1, fh), W2, b2.reshape(ne, 1, D))

    yg = _sc_gather(y, pair_pos, 32)

    blk6 = min(512, BT)
    n6 = BT // blk6
    out = pl.pallas_call(
        _k6,
        grid=(n6,),
        in_specs=[
            pl.BlockSpec((blk6, D), lambda i: (i, 0)),
            pl.BlockSpec((blk6, D), lambda i: (i, 0)),
            pl.BlockSpec((blk6, D), lambda i: (i + BT // blk6, 0)),
            pl.BlockSpec((blk6, NA), lambda i: (i, 0)),
        ],
        out_specs=pl.BlockSpec((blk6, D), lambda i: (i, 0)),
        out_shape=jax.ShapeDtypeStruct((BT, D), f32),
    )(x2, yg, yg, wcol)

    f_i = jnp.sum(csum, axis=(0, 1)) / (BT * NA)
    P_i = jnp.sum(psum, axis=(0, 1)) / BT
    aux = ne * jnp.sum(f_i * P_i)
    return out.reshape(B, T, D), new_state, aux


# speedup vs baseline: 1.2108x; 1.2108x over previous
import functools

import jax
import jax.numpy as jnp
from jax import lax
from jax.experimental import pallas as pl
from jax.experimental.pallas import tpu as pltpu
from jax.experimental.pallas import tpu_sc as plsc

DM = 1024
DS = 64
NE = 8
NA = 2
FH = 2048
CK = 64
EPS = 1e-8
GBLK = 256


def _compnorm(x, tau, scale):
    rms = jax.lax.rsqrt(jnp.mean(x * x, axis=-1, keepdims=True) + EPS)
    x_norm = x * rms
    xc = x - jnp.mean(x, axis=-1, keepdims=True)
    gate = jax.nn.softmax(xc / jnp.maximum(tau, 1.0), axis=-1)
    return x_norm * gate * scale * DM


def _k1(x_ref, tau_ref, scale_ref, wa_ref, wg_ref, ba_ref, bg_ref, u_ref):
    x = x_ref[...]
    h = _compnorm(x, tau_ref[0, 0], scale_ref[...])
    iv = jnp.tanh(jnp.dot(h, wa_ref[...], preferred_element_type=jnp.float32)
                  + ba_ref[...])
    g = jax.nn.sigmoid(jnp.dot(h, wg_ref[...], preferred_element_type=jnp.float32)
                       + bg_ref[...])
    u_ref[...] = g * iv


def _k2(ut_ref, u_ref, rec_ref, pos_ref, state_ref,
        fu_ref, hs_ref, ns_ref, r_scr, *, nb):
    la_col = jnp.log(jax.nn.sigmoid(rec_ref[...] + pos_ref[...]))
    la_row = la_col.reshape(1, DS)
    i_col = jax.lax.broadcasted_iota(jnp.int32, (CK, 1), 0).astype(jnp.float32)
    j2 = jax.lax.broadcasted_iota(jnp.int32, (CK, CK), 0).astype(jnp.float32)
    i2 = jax.lax.broadcasted_iota(jnp.int32, (CK, CK), 1).astype(jnp.float32)
    expn = jnp.maximum(j2 - i2, 0.0)
    mask = (j2 >= i2).astype(jnp.float32)
    L = jnp.exp(la_col.reshape(DS, 1, 1) * expn[None]) * mask[None]
    fu = jax.lax.dot_general(L, ut_ref[...],
                             (((2,), (1,)), ((0,), (0,))),
                             preferred_element_type=jnp.float32)
    fu_ref[...] = fu
    w = jnp.exp(la_row * (CK - 1.0 - i_col))
    r_scr[...] = jnp.sum(u_ref[...] * w[None], axis=1)
    a_ck = jnp.exp(la_row * float(CK))

    bsz = state_ref.shape[0]

    def body(c, h_cur):
        for b in range(bsz):
            hs_ref[pl.ds(b * nb + c, 1), :] = h_cur[b:b + 1]
        rows = jnp.concatenate(
            [r_scr[pl.ds(b * nb + c, 1), :] for b in range(bsz)], axis=0)
        return h_cur * a_ck + rows

    h_fin = jax.lax.fori_loop(0, nb, body, state_ref[...])
    ns_ref[...] = h_fin


def _k3(fu_ref, hs_ref, ap_ref, x_ref, tau_ref, scale_ref, wout_ref,
        bout_ref, wr_ref, x2_ref, h3_ref, io_ref, wc_ref, ps_ref, cs_ref):
    states = fu_ref[...] + hs_ref[...] * ap_ref[...]
    h2 = jnp.dot(states, wout_ref[...], preferred_element_type=jnp.float32) \
        + bout_ref[...]
    x2 = x_ref[...] + h2
    x2_ref[...] = x2
    h3 = _compnorm(x2, tau_ref[0, 0], scale_ref[...])
    h3_ref[...] = h3
    logits = jnp.dot(h3, wr_ref[...], preferred_element_type=jnp.float32)
    probs = jax.nn.softmax(logits, axis=-1)
    eio = jax.lax.broadcasted_iota(jnp.int32, logits.shape, 1)
    m1 = jnp.max(logits, axis=-1, keepdims=True)
    idx1 = jnp.min(jnp.where(logits == m1, eio, NE), axis=-1, keepdims=True)
    mask1 = (eio == idx1)
    ml = jnp.where(mask1, -jnp.inf, logits)
    m2 = jnp.max(ml, axis=-1, keepdims=True)
    idx2 = jnp.min(jnp.where(ml == m2, eio, NE), axis=-1, keepdims=True)
    mask2 = (eio == idx2)
    p1 = jnp.sum(jnp.where(mask1, probs, 0.0), axis=-1, keepdims=True)
    p2 = jnp.sum(jnp.where(mask2, probs, 0.0), axis=-1, keepdims=True)
    inv = 1.0 / (p1 + p2)
    two = jax.lax.broadcasted_iota(jnp.int32, (logits.shape[0], 2), 1)
    io_ref[...] = jnp.where(two == 0, idx1, idx2)
    wc_ref[...] = jnp.where(two == 0, p1 * inv, p2 * inv)
    ps_ref[...] = jnp.sum(probs, axis=0, keepdims=True)[None]
    cs_ref[...] = jnp.sum(mask1.astype(jnp.float32) + mask2.astype(jnp.float32),
                          axis=0, keepdims=True)[None]


def _sc_gather(table, idx, rpc):
    sci = plsc.get_sparse_core_info()
    ncores = sci.num_cores
    nw = ncores * sci.num_subcores
    n = idx.shape[0]
    d = table.shape[1]
    b_per_w = n // nw
    nchunk = b_per_w // rpc
    assert nchunk % 2 == 0
    mesh = plsc.VectorSubcoreMesh(core_axis_name="c", subcore_axis_name="s")

    @functools.partial(
        pl.kernel, mesh=mesh,
        out_type=jax.ShapeDtypeStruct((n, d), table.dtype),
        scratch_types=[
            pltpu.VMEM((b_per_w,), jnp.int32),
            pltpu.VMEM((2, rpc, d), table.dtype),
            pltpu.SemaphoreType.DMA,
            pltpu.SemaphoreType.DMA,
        ],
    )
    def k(table_hbm, idx_hbm, out_hbm, idx_v, rows_v, sem0, sem1):
        wid = lax.axis_index("s") * ncores + lax.axis_index("c")
        base = wid * b_per_w
        cbase = wid * nchunk
        pltpu.sync_copy(idx_hbm.at[pl.ds(base, b_per_w)], idx_v)
        sems = (sem0, sem1)

        def gather(cc, b):
            off = pl.multiple_of(cc * rpc, rpc)
            return pltpu.make_async_copy(
                table_hbm.at[idx_v.at[pl.ds(off, rpc)]],
                rows_v.at[b], sems[b])

        gather(0, 0).start()

        @pl.loop(0, nchunk, step=2)
        def body(c):
            for b in range(2):
                cc = c + b
                gather(cc, b).wait()

                @pl.when(cc + 1 < nchunk)
                def _():
                    gather(cc + 1, 1 - b).start()

                pltpu.sync_copy(rows_v.at[b],
                                out_hbm.at[pl.ds((cbase + cc) * rpc, rpc)])

    return k(table, idx)


def _k5(be_ref, xg_ref, w1_ref, b1_ref, w2_ref, b2_ref, y_ref):
    h = jax.nn.silu(jnp.dot(xg_ref[...], w1_ref[0],
                            preferred_element_type=jnp.float32) + b1_ref[0])
    y_ref[...] = jnp.dot(h, w2_ref[0],
                         preferred_element_type=jnp.float32) + b2_ref[0]


def _k6(x2_ref, ya_ref, yb_ref, wc_ref, out_ref):
    w = wc_ref[...]
    out_ref[...] = (x2_ref[...] + w[:, 0:1] * ya_ref[...]
                    + w[:, 1:2] * yb_ref[...])


def kernel(x, state, tau1, scale1, tau2, scale2, Win, bin_, rec_w, pos_bias,
           Wout, bout, Wr, W1, b1, W2, b2):
    B, T, D = x.shape
    ds = Wout.shape[0]
    ne = Wr.shape[1]
    fh = W1.shape[2]
    BT = B * T
    nch = T // CK
    BC = B * nch
    xf = x.reshape(BT, D)
    f32 = jnp.float32

    blk1 = min(512, BT)
    n1 = BT // blk1
    u = pl.pallas_call(
        _k1,
        grid=(n1,),
        in_specs=[
            pl.BlockSpec((blk1, D), lambda i: (i, 0)),
            pl.BlockSpec((1, 1), lambda i: (0, 0)),
            pl.BlockSpec((1, D), lambda i: (0, 0)),
            pl.BlockSpec((D, ds), lambda i: (0, 0)),
            pl.BlockSpec((D, ds), lambda i: (0, 0)),
            pl.BlockSpec((1, ds), lambda i: (0, 0)),
            pl.BlockSpec((1, ds), lambda i: (0, 0)),
        ],
        out_specs=pl.BlockSpec((blk1, ds), lambda i: (i, 0)),
        out_shape=jax.ShapeDtypeStruct((BT, ds), f32),
    )(xf, tau1.reshape(1, 1), scale1.reshape(1, D), Win[:, :ds], Win[:, ds:],
      bin_[:ds].reshape(1, ds), bin_[ds:].reshape(1, ds))

    u4 = u.reshape(B, nch, CK, ds)
    ut = jnp.transpose(u4, (3, 2, 0, 1)).reshape(ds, CK, BC)
    uo = u4.reshape(BC, CK, ds)

    fu, hs, new_state = pl.pallas_call(
        functools.partial(_k2, nb=nch),
        grid=(1,),
        in_specs=[
            pl.BlockSpec((ds, CK, BC), lambda i: (0, 0, 0)),
            pl.BlockSpec((BC, CK, ds), lambda i: (0, 0, 0)),
            pl.BlockSpec((ds, 1), lambda i: (0, 0)),
            pl.BlockSpec((ds, 1), lambda i: (0, 0)),
            pl.BlockSpec((B, ds), lambda i: (0, 0)),
        ],
        out_specs=[
            pl.BlockSpec((ds, CK, BC), lambda i: (0, 0, 0)),
            pl.BlockSpec((BC, ds), lambda i: (0, 0)),
            pl.BlockSpec((B, ds), lambda i: (0, 0)),
        ],
        out_shape=[
            jax.ShapeDtypeStruct((ds, CK, BC), f32),
            jax.ShapeDtypeStruct((BC, ds), f32),
            jax.ShapeDtypeStruct((B, ds), f32),
        ],
        scratch_shapes=[pltpu.VMEM((BC, ds), f32)],
    )(ut, uo, rec_w.reshape(ds, 1), pos_bias.reshape(ds, 1), state)

    fu_t = jnp.transpose(fu, (2, 1, 0)).reshape(BT, ds)
    hs_full = jnp.repeat(hs, CK, axis=0)
    a = jax.nn.sigmoid(rec_w + pos_bias)
    jj = (jnp.arange(CK, dtype=f32) + 1.0)[:, None]
    apow = a[None, :] ** jj
    ap_full = jnp.tile(apow, (BC, 1))

    blk3 = min(512, BT)
    n3 = BT // blk3
    x2, h3, iout, wcol, psum, csum = pl.pallas_call(
        _k3,
        grid=(n3,),
        in_specs=[
            pl.BlockSpec((blk3, ds), lambda i: (i, 0)),
            pl.BlockSpec((blk3, ds), lambda i: (i, 0)),
            pl.BlockSpec((blk3, ds), lambda i: (i, 0)),
            pl.BlockSpec((blk3, D), lambda i: (i, 0)),
            pl.BlockSpec((1, 1), lambda i: (0, 0)),
            pl.BlockSpec((1, D), lambda i: (0, 0)),
            pl.BlockSpec((ds, D), lambda i: (0, 0)),
            pl.BlockSpec((1, D), lambda i: (0, 0)),
            pl.BlockSpec((D, ne), lambda i: (0, 0)),
        ],
        out_specs=[
            pl.BlockSpec((blk3, D), lambda i: (i, 0)),
            pl.BlockSpec((blk3, D), lambda i: (i, 0)),
            pl.BlockSpec((blk3, NA), lambda i: (i, 0)),
            pl.BlockSpec((blk3, NA), lambda i: (i, 0)),
            pl.BlockSpec((1, 1, ne), lambda i: (i, 0, 0)),
            pl.BlockSpec((1, 1, ne), lambda i: (i, 0, 0)),
        ],
        out_shape=[
            jax.ShapeDtypeStruct((BT, D), f32),
            jax.ShapeDtypeStruct((BT, D), f32),
            jax.ShapeDtypeStruct((BT, NA), jnp.int32),
            jax.ShapeDtypeStruct((BT, NA), f32),
            jax.ShapeDtypeStruct((n3, 1, ne), f32),
            jax.ShapeDtypeStruct((n3, 1, ne), f32),
        ],
    )(fu_t, hs_full, ap_full, xf, tau2.reshape(1, 1), scale2.reshape(1, D),
      Wout, bout.reshape(1, D), Wr)

    npair = BT * NA
    nblk = npair // GBLK + NE
    npad = nblk * GBLK
    ep = iout.reshape(-1)
    perm = jnp.argsort(ep, stable=True).astype(jnp.int32)
    perm_inv = jnp.argsort(perm).astype(jnp.int32)
    counts = jnp.sum((ep[:, None] == jnp.arange(ne)[None, :]), axis=0)
    nbe = (counts + GBLK - 1) // GBLK
    nbc = jnp.cumsum(nbe)
    pstart = ((nbc - nbe) * GBLK).astype(jnp.int32)
    cstart = (jnp.cumsum(counts) - counts).astype(jnp.int32)
    block_expert = jnp.minimum(
        jnp.searchsorted(nbc, jnp.arange(nblk), side='right'),
        ne - 1).astype(jnp.int32)
    pos_of_pair = pstart[ep] + perm_inv - cstart[ep]
    pair_pos = jnp.concatenate([pos_of_pair[0::2], pos_of_pair[1::2]])
    rr = jnp.arange(npad, dtype=jnp.int32)
    be_r = block_expert[rr // GBLK]
    kk = rr - pstart[be_r]
    jj_r = jnp.clip(cstart[be_r] + kk, 0, npair - 1)
    tok_for_row = jnp.where(kk < counts[be_r], perm[jj_r] // NA, rr % BT)

    xg = _sc_gather(h3, tok_for_row, 32)

    grid_spec = pltpu.PrefetchScalarGridSpec(
        num_scalar_prefetch=1,
        grid=(nblk,),
        in_specs=[
            pl.BlockSpec((GBLK, D), lambda i, be: (i, 0)),
            pl.BlockSpec((1, D, fh), lambda i, be: (be[i], 0, 0)),
            pl.BlockSpec((1, 1, fh), lambda i, be: (be[i], 0, 0)),
            pl.BlockSpec((1, fh, D), lambda i, be: (be[i], 0, 0)),
            pl.BlockSpec((1, 1, D), lambda i, be: (be[i], 0, 0)),
        ],
        out_specs=pl.BlockSpec((GBLK, D), lambda i, be: (i, 0)),
    )
    y = pl.pallas_call(
        _k5,
        grid_spec=grid_spec,
        out_shape=jax.ShapeDtypeStruct((npad, D), f32),
    )(block_expert, xg, W1, b1.reshape(ne, 1, fh), W2, b2.reshape(ne, 1, D))

    yg = _sc_gather(y, pair_pos, 32)

    blk6 = min(512, BT)
    n6 = BT // blk6
    out = pl.pallas_call(
        _k6,
        grid=(n6,),
        in_specs=[
            pl.BlockSpec((blk6, D), lambda i: (i, 0)),
            pl.BlockSpec((blk6, D), lambda i: (i, 0)),
            pl.BlockSpec((blk6, D), lambda i: (i + BT // blk6, 0)),
            pl.BlockSpec((blk6, NA), lambda i: (i, 0)),
        ],
        out_specs=pl.BlockSpec((blk6, D), lambda i: (i, 0)),
        out_shape=jax.ShapeDtypeStruct((BT, D), f32),
    )(x2, yg, yg, wcol)

    f_i = jnp.sum(csum, axis=(0, 1)) / (BT * NA)
    P_i = jnp.sum(psum, axis=(0, 1)) / BT
    aux = ne * jnp.sum(f_i * P_i)
    return out.reshape(B, T, D), new_state, aux
